# baseline (speedup 1.0000x reference)
import functools

import jax
import jax.numpy as jnp
from jax import lax
from jax.experimental import pallas as pl
from jax.experimental.pallas import tpu as pltpu
from jax.experimental.pallas import tpu_sc as plsc

_E = 3_200_000
_D = 16
_NC = 2
_NS = 16
_NW = _NC * _NS
_B = 640
_NBLK = _E // _B
_L = 16


def _sc_body(edge_hbm, table_hbm, out_hbm, table_v, ij_v, rows_v,
             isem0, isem1, osem0, osem1):
    wid = lax.axis_index("s") * _NC + lax.axis_index("c")
    isems = (isem0, isem1)
    osems = (osem0, osem1)
    pltpu.sync_copy(table_hbm, table_v)

    def start_in(i, h):
        g = wid + i * _NW

        @pl.when(g < _NBLK)
        def _():
            base = g * _B
            pltpu.async_copy(edge_hbm.at[:, pl.ds(base, _B)], ij_v.at[h],
                             isems[h])

    def wait_in(h):
        pltpu.make_async_copy(edge_hbm.at[:, pl.ds(0, _B)], ij_v.at[h],
                              isems[h]).wait()

    def wait_out(h):
        pltpu.make_async_copy(rows_v.at[h], out_hbm.at[:, pl.ds(0, _B)],
                              osems[h]).wait()

    start_in(0, 0)
    start_in(1, 1)

    def pair(k, carry):
        for h in (0, 1):
            i = 2 * k + h
            g = wid + i * _NW

            @pl.when(g < _NBLK)
            def _():
                base = g * _B
                wait_in(h)

                @pl.when(k > 0)
                def _():
                    wait_out(h)

                @plsc.parallel_loop(0, _B // _L, 1, unroll=2)
                def grp(j):
                    o = j * _L
                    a = jnp.abs(ij_v[h, 0, pl.ds(o, _L)] -
                                ij_v[h, 1, pl.ds(o, _L)])
                    t = jnp.maximum(a - 1, 0).astype(jnp.float32)
                    bits = lax.bitcast_convert_type(t, jnp.int32)
                    codes = jnp.maximum((bits >> 23) - 126, 0)
                    for d in range(_D):
                        rows_v[h, d, pl.ds(o, _L)] = plsc.load_gather(
                            table_v, [codes + (d * 128)])

                pltpu.async_copy(rows_v.at[h], out_hbm.at[:, pl.ds(base, _B)],
                                 osems[h])
                start_in(i + 2, h)
        return carry

    lax.fori_loop(0, (_NBLK // _NW + 2) // 2, pair, 0)
    wait_out(0)
    wait_out(1)


def kernel(edge_index, emb_weight):
    table_t = jnp.zeros((_D, 128), jnp.float32).at[:, :18].set(emb_weight.T)
    run = pl.kernel(
        _sc_body,
        out_type=jax.ShapeDtypeStruct((_D, _E), jnp.float32),
        mesh=plsc.VectorSubcoreMesh(core_axis_name="c", subcore_axis_name="s"),
        compiler_params=pltpu.CompilerParams(
            use_tc_tiling_on_sc=True, needs_layout_passes=False),
        scratch_types=[
            pltpu.VMEM((_D * 128,), jnp.float32),
            pltpu.VMEM((2, 2, _B), jnp.int32),
            pltpu.VMEM((2, _D, _B), jnp.float32),
            pltpu.SemaphoreType.DMA,
            pltpu.SemaphoreType.DMA,
            pltpu.SemaphoreType.DMA,
            pltpu.SemaphoreType.DMA,
        ],
    )
    out_t = run(edge_index, table_t.reshape(-1))
    return out_t.T


# speedup vs baseline: 1.0251x; 1.0251x over previous
import functools

import jax
import jax.numpy as jnp
from jax import lax
from jax.experimental import pallas as pl
from jax.experimental.pallas import tpu as pltpu
from jax.experimental.pallas import tpu_sc as plsc

_E = 3_200_000
_D = 16
_NC = 2
_NS = 16
_NW = _NC * _NS
_B = 1_024
_NBLK = _E // _B
_L = 16


def _sc_body(edge_hbm, table_hbm, out_hbm, table_v, ij_v, rows_v,
             isem0, isem1, osem0, osem1):
    wid = lax.axis_index("s") * _NC + lax.axis_index("c")
    isems = (isem0, isem1)
    osems = (osem0, osem1)
    pltpu.sync_copy(table_hbm, table_v)

    def start_in(i, h):
        g = wid + i * _NW

        @pl.when(g < _NBLK)
        def _():
            base = g * _B
            pltpu.async_copy(edge_hbm.at[:, pl.ds(base, _B)], ij_v.at[h],
                             isems[h])

    def wait_in(h):
        pltpu.make_async_copy(edge_hbm.at[:, pl.ds(0, _B)], ij_v.at[h],
                              isems[h]).wait()

    def wait_out(h):
        pltpu.make_async_copy(rows_v.at[h], out_hbm.at[:, pl.ds(0, _B)],
                              osems[h]).wait()

    start_in(0, 0)
    start_in(1, 1)

    def pair(k, carry):
        for h in (0, 1):
            i = 2 * k + h
            g = wid + i * _NW

            @pl.when(g < _NBLK)
            def _():
                base = g * _B
                wait_in(h)

                @pl.when(k > 0)
                def _():
                    wait_out(h)

                @plsc.parallel_loop(0, _B // _L, 1, unroll=2)
                def grp(j):
                    o = j * _L
                    a = jnp.abs(ij_v[h, 0, pl.ds(o, _L)] -
                                ij_v[h, 1, pl.ds(o, _L)])
                    t = jnp.maximum(a - 1, 0).astype(jnp.float32)
                    bits = lax.bitcast_convert_type(t, jnp.int32)
                    codes = jnp.maximum((bits >> 23) - 126, 0)
                    for d in range(_D):
                        rows_v[h, d, pl.ds(o, _L)] = plsc.load_gather(
                            table_v, [codes + (d * 128)])

                pltpu.async_copy(rows_v.at[h], out_hbm.at[:, pl.ds(base, _B)],
                                 osems[h])
                start_in(i + 2, h)
        return carry

    lax.fori_loop(0, (_NBLK // _NW + 2) // 2, pair, 0)
    wait_out(0)
    wait_out(1)


def kernel(edge_index, emb_weight):
    table_t = jnp.zeros((_D, 128), jnp.float32).at[:, :18].set(emb_weight.T)
    run = pl.kernel(
        _sc_body,
        out_type=jax.ShapeDtypeStruct((_D, _E), jnp.float32),
        mesh=plsc.VectorSubcoreMesh(core_axis_name="c", subcore_axis_name="s"),
        compiler_params=pltpu.CompilerParams(
            use_tc_tiling_on_sc=True, needs_layout_passes=False),
        scratch_types=[
            pltpu.VMEM((_D * 128,), jnp.float32),
            pltpu.VMEM((2, 2, _B), jnp.int32),
            pltpu.VMEM((2, _D, _B), jnp.float32),
            pltpu.SemaphoreType.DMA,
            pltpu.SemaphoreType.DMA,
            pltpu.SemaphoreType.DMA,
            pltpu.SemaphoreType.DMA,
        ],
    )
    out_t = run(edge_index, table_t.reshape(-1))
    return out_t.T


# speedup vs baseline: 1.0936x; 1.0669x over previous
import jax
import jax.numpy as jnp
from jax import lax
from jax.experimental import pallas as pl
from jax.experimental.pallas import tpu as pltpu
from jax.experimental.pallas import tpu_sc as plsc

_E = 3_200_000
_D = 16
_NC = 2
_NS = 16
_NW = _NC * _NS
_B = 1_280
_NBLK = _E // _B
_L = 16


def _sc_body(edge_hbm, table_hbm, out_hbm, table_v, ij_v, rows_v,
             isem0, isem1, osem0, osem1):
    wid = lax.axis_index("s") * _NC + lax.axis_index("c")
    isems = (isem0, isem1)
    osems = (osem0, osem1)
    pltpu.sync_copy(table_hbm, table_v)

    def start_in(i, h):
        g = wid + i * _NW

        @pl.when(g < _NBLK)
        def _():
            base = g * _B
            pltpu.async_copy(edge_hbm.at[:, pl.ds(base, _B)], ij_v.at[h],
                             isems[h])

    def wait_in(h):
        pltpu.make_async_copy(edge_hbm.at[:, pl.ds(0, _B)], ij_v.at[h],
                              isems[h]).wait()

    def wait_out(h):
        pltpu.make_async_copy(rows_v.at[h], out_hbm.at[:, pl.ds(0, _B)],
                              osems[h]).wait()

    start_in(0, 0)
    start_in(1, 1)

    def pair(k, carry):
        for h in (0, 1):
            i = 2 * k + h
            g = wid + i * _NW

            @pl.when(g < _NBLK)
            def _():
                base = g * _B
                wait_in(h)

                @pl.when(k > 0)
                def _():
                    wait_out(h)

                @plsc.parallel_loop(0, _B // _L, 1, unroll=2)
                def grp(j):
                    o = j * _L
                    a = jnp.abs(ij_v[h, 0, pl.ds(o, _L)] -
                                ij_v[h, 1, pl.ds(o, _L)])
                    t = jnp.maximum(a - 1, 0).astype(jnp.float32)
                    bits = lax.bitcast_convert_type(t, jnp.int32)
                    codes = jnp.maximum((bits >> 23) - 126, 0)
                    for d in range(_D):
                        rows_v[h, d, pl.ds(o, _L)] = plsc.load_gather(
                            table_v, [codes + (d * 128)])

                pltpu.async_copy(rows_v.at[h], out_hbm.at[:, pl.ds(base, _B)],
                                 osems[h])
                start_in(i + 2, h)
        return carry

    lax.fori_loop(0, (_NBLK // _NW + 2) // 2, pair, 0)
    wait_out(0)
    wait_out(1)


def kernel(edge_index, emb_weight):
    table_t = jnp.zeros((_D, 128), jnp.float32).at[:, :18].set(emb_weight.T)
    run = pl.kernel(
        _sc_body,
        out_type=jax.ShapeDtypeStruct((_D, _E), jnp.float32),
        mesh=plsc.VectorSubcoreMesh(core_axis_name="c", subcore_axis_name="s"),
        compiler_params=pltpu.CompilerParams(
            use_tc_tiling_on_sc=True, needs_layout_passes=False),
        scratch_types=[
            pltpu.VMEM((_D * 128,), jnp.float32),
            pltpu.VMEM((2, 2, _B), jnp.int32),
            pltpu.VMEM((2, _D, _B), jnp.float32),
            pltpu.SemaphoreType.DMA,
            pltpu.SemaphoreType.DMA,
            pltpu.SemaphoreType.DMA,
            pltpu.SemaphoreType.DMA,
        ],
    )
    out_t = run(edge_index, table_t.reshape(-1))
    return out_t.T


# speedup vs baseline: 1.1144x; 1.0189x over previous
import jax
import jax.numpy as jnp
from jax import lax
from jax.experimental import pallas as pl
from jax.experimental.pallas import tpu as pltpu
from jax.experimental.pallas import tpu_sc as plsc

_E = 3_200_000
_D = 16
_NC = 2
_NS = 16
_NW = _NC * _NS
_B = 1_280
_NBLK = _E // _B
_L = 16


def _sc_body(edge_hbm, table_hbm, out_hbm, table_v, ij_v, rows_v,
             isem0, isem1, osem0, osem1):
    wid = lax.axis_index("s") * _NC + lax.axis_index("c")
    isems = (isem0, isem1)
    osems = (osem0, osem1)
    pltpu.sync_copy(table_hbm, table_v)

    def start_in(i, h):
        g = wid + i * _NW

        @pl.when(g < _NBLK)
        def _():
            base = g * _B
            pltpu.async_copy(edge_hbm.at[:, pl.ds(base, _B)], ij_v.at[h],
                             isems[h])

    def wait_in(h):
        pltpu.make_async_copy(edge_hbm.at[:, pl.ds(0, _B)], ij_v.at[h],
                              isems[h]).wait()

    def wait_out(h):
        pltpu.make_async_copy(rows_v.at[h], out_hbm.at[:, pl.ds(0, _B)],
                              osems[h]).wait()

    start_in(0, 0)
    start_in(1, 1)

    def pair(k, carry):
        for h in (0, 1):
            i = 2 * k + h
            g = wid + i * _NW

            @pl.when(g < _NBLK)
            def _():
                base = g * _B
                wait_in(h)

                @pl.when(k > 0)
                def _():
                    wait_out(h)

                @plsc.parallel_loop(0, _B // _L, 1, unroll=3)
                def grp(j):
                    o = j * _L
                    a = jnp.abs(ij_v[h, 0, pl.ds(o, _L)] -
                                ij_v[h, 1, pl.ds(o, _L)])
                    t = jnp.maximum(a - 1, 0).astype(jnp.float32)
                    bits = lax.bitcast_convert_type(t, jnp.int32)
                    codes = jnp.maximum((bits >> 23) - 126, 0)
                    for d in range(_D):
                        rows_v[h, d, pl.ds(o, _L)] = plsc.load_gather(
                            table_v, [codes + (d * 128)])

                pltpu.async_copy(rows_v.at[h], out_hbm.at[:, pl.ds(base, _B)],
                                 osems[h])
                start_in(i + 2, h)
        return carry

    lax.fori_loop(0, (_NBLK // _NW + 2) // 2, pair, 0)
    wait_out(0)
    wait_out(1)


def kernel(edge_index, emb_weight):
    table_t = jnp.zeros((_D, 128), jnp.float32).at[:, :18].set(emb_weight.T)
    run = pl.kernel(
        _sc_body,
        out_type=jax.ShapeDtypeStruct((_D, _E), jnp.float32),
        mesh=plsc.VectorSubcoreMesh(core_axis_name="c", subcore_axis_name="s"),
        compiler_params=pltpu.CompilerParams(
            use_tc_tiling_on_sc=True, needs_layout_passes=False),
        scratch_types=[
            pltpu.VMEM((_D * 128,), jnp.float32),
            pltpu.VMEM((2, 2, _B), jnp.int32),
            pltpu.VMEM((2, _D, _B), jnp.float32),
            pltpu.SemaphoreType.DMA,
            pltpu.SemaphoreType.DMA,
            pltpu.SemaphoreType.DMA,
            pltpu.SemaphoreType.DMA,
        ],
    )
    out_t = run(edge_index, table_t.reshape(-1))
    return out_t.T
